# baseline (speedup 1.0000x reference)
import functools

import jax
import jax.numpy as jnp
import numpy as np
from jax import lax
from jax.experimental import pallas as pl
from jax.experimental.pallas import tpu as pltpu
from jax.experimental.pallas import tpu_sc as plsc

RANGE_MIN = -5.0
RANGE_MAX = 5.0
MIN_BIN_SIZE = 0.0001
MIN_SLOPE = 0.0001

LN2 = 0.6931471805599453
SQRT2 = 1.4142135623730951

N = 8388608
NC, NS, L = 2, 16, 16
NW = NC * NS
PER_W = N // NW
CHUNK = 16384
N_CHUNKS = PER_W // CHUNK
N_STEPS = N_CHUNKS // 2
VREGS = CHUNK // L
TPAD = 144


def _log_approx(t):
    bits = lax.bitcast_convert_type(t, jnp.int32)
    e_i = (bits >> 23) - 127
    m = lax.bitcast_convert_type((bits & 0x007FFFFF) | 0x3F800000, jnp.float32)
    big = m >= SQRT2
    m = jnp.where(big, m * 0.5, m)
    e_f = e_i.astype(jnp.float32) + jnp.where(big, 1.0, 0.0)
    z = (m - 1.0) / (m + 1.0)
    z2 = z * z
    p = z * (2.0 + z2 * (2.0 / 3.0 + z2 * (2.0 / 5.0 + z2 * (2.0 / 7.0))))
    return e_f * LN2 + p


def _sc_body(x_hbm, xpos_hbm, ypos_hbm, d_hbm, invw_hbm, h_hbm, t0_hbm,
             xposc_hbm, c0_hbm, ln_hbm, rcp_hbm,
             y_hbm, ld_hbm,
             xpos_v, ypos_v, d_v, invw_v, h_v, t0_v, xposc_v, c0_v, ln_v, rcp_v,
             x0, x1, y0, y1, l0, l1,
             sem_in0, sem_in1, sem_oy0, sem_oy1, sem_ol0, sem_ol1):
    wid = lax.axis_index("s") * NC + lax.axis_index("c")
    base = wid * PER_W

    pltpu.sync_copy(xpos_hbm, xpos_v)
    pltpu.sync_copy(ypos_hbm, ypos_v)
    pltpu.sync_copy(d_hbm, d_v)
    pltpu.sync_copy(invw_hbm, invw_v)
    pltpu.sync_copy(h_hbm, h_v)
    pltpu.sync_copy(t0_hbm, t0_v)
    pltpu.sync_copy(xposc_hbm, xposc_v)
    pltpu.sync_copy(c0_hbm, c0_v)
    pltpu.sync_copy(ln_hbm, ln_v)
    pltpu.sync_copy(rcp_hbm, rcp_v)

    coarse = xposc_v[pl.ds(0, L)]

    zero_f = jnp.zeros((L,), jnp.float32)
    one_f = jnp.full((L,), 1.0, jnp.float32)
    rmin_f = jnp.full((L,), RANGE_MIN, jnp.float32)
    rmax_f = jnp.full((L,), RANGE_MAX, jnp.float32)
    ln2_f = jnp.full((L,), LN2, jnp.float32)
    zero_i = jnp.zeros((L,), jnp.int32)
    m7f_i = jnp.full((L,), 0x7F, jnp.int32)
    mffff_i = jnp.full((L,), 0xFFFF, jnp.int32)

    def make_vreg_body(x_v, y_v, ld_v):
      def vreg_body(off):
        xv = x_v[pl.ds(off, L)]
        c = zero_i
        for step in (8, 4, 2, 1):
            cand = c + step
            knot = jnp.take_along_axis(coarse, cand, axis=0)
            c = jnp.where(knot <= xv, cand, c)
        b = c * 8
        for step in (4, 2, 1):
            cand = b + step
            knot = plsc.load_gather(xpos_v, [cand])
            b = jnp.where(knot <= xv, cand, b)
        t0 = plsc.load_gather(t0_v, [b])
        y_k = plsc.load_gather(ypos_v, [b])
        iw = plsc.load_gather(invw_v, [b])
        hh = plsc.load_gather(h_v, [b])
        d_k = plsc.load_gather(d_v, [b])
        d_k1 = plsc.load_gather(d_v, [b + 1])
        c0 = plsc.load_gather(c0_v, [b])
        s_ = hh * iw
        xi = jnp.minimum(jnp.maximum(xv * iw + t0, zero_f), one_f)
        om = one_f - xi
        xiom = xi * om
        dkom = d_k * om
        num = xi * (s_ * xi + dkom)
        den = s_ + c0 * xiom
        rden = 1.0 / den
        y_spline = y_k + hh * (num * rden)
        numd = s_ * s_ * (d_k1 * xi * xi + (s_ + s_) * xiom + dkom * om)
        deriv = numd * (rden * rden)
        below = xv < rmin_f
        above = xv > rmax_f
        yv = jnp.where(below, (xv - rmin_f) * d_k + rmin_f,
                       jnp.where(above, (xv - rmax_f) * d_k1 + rmax_f,
                                 y_spline))
        bits = lax.bitcast_convert_type(deriv, jnp.int32)
        ubits = lax.bitcast_convert_type(deriv, jnp.uint32)
        e_f = (bits >> 23).astype(jnp.float32)
        j = lax.convert_element_type((ubits << 9) >> 25, jnp.int32)
        f_cvt = ((ubits << 16) >> 16).astype(jnp.float32)
        lnm = plsc.load_gather(ln_v, [j]) + f_cvt * plsc.load_gather(rcp_v, [j])
        y_v[pl.ds(off, L)] = yv
        ld_v[pl.ds(off, L)] = e_f * ln2_f + lnm
      return vreg_body

    def half(i, g, x_v, y_v, ld_v, sem_in, sem_oy, sem_ol):
        lo = base + g * CHUNK
        out_y = pltpu.make_async_copy(y_v, y_hbm.at[pl.ds(lo, CHUNK)], sem_oy)
        out_l = pltpu.make_async_copy(ld_v, ld_hbm.at[pl.ds(lo, CHUNK)], sem_ol)

        @pl.when(i > 0)
        def _():
            out_y.wait()
            out_l.wait()

        pltpu.make_async_copy(x_hbm.at[pl.ds(lo, CHUNK)], x_v, sem_in).wait()
        plsc.parallel_loop(0, CHUNK, L, unroll=20)(make_vreg_body(x_v, y_v, ld_v))
        out_y.start()
        out_l.start()

        @pl.when(i < N_STEPS - 1)
        def _():
            nxt = lo + 2 * CHUNK
            pltpu.make_async_copy(x_hbm.at[pl.ds(nxt, CHUNK)], x_v, sem_in).start()

    pltpu.make_async_copy(x_hbm.at[pl.ds(base, CHUNK)], x0, sem_in0).start()
    pltpu.make_async_copy(x_hbm.at[pl.ds(base + CHUNK, CHUNK)], x1, sem_in1).start()

    def step(i, carry):
        half(i, 2 * i, x0, y0, l0, sem_in0, sem_oy0, sem_ol0)
        half(i, 2 * i + 1, x1, y1, l1, sem_in1, sem_oy1, sem_ol1)
        return carry

    lax.fori_loop(0, N_STEPS, step, 0)

    tail = base + (N_CHUNKS - 2) * CHUNK
    pltpu.make_async_copy(y0, y_hbm.at[pl.ds(tail, CHUNK)], sem_oy0).wait()
    pltpu.make_async_copy(l0, ld_hbm.at[pl.ds(tail, CHUNK)], sem_ol0).wait()
    pltpu.make_async_copy(y1, y_hbm.at[pl.ds(tail + CHUNK, CHUNK)], sem_oy1).wait()
    pltpu.make_async_copy(l1, ld_hbm.at[pl.ds(tail + CHUNK, CHUNK)], sem_ol1).wait()


@jax.jit
def kernel(x, params):
    K = (params.shape[-1] - 1) // 3
    total_size = RANGE_MAX - RANGE_MIN
    widths = jax.nn.softmax(params[:K]) * (total_size - K * MIN_BIN_SIZE) + MIN_BIN_SIZE
    heights = jax.nn.softmax(params[K:2 * K]) * (total_size - K * MIN_BIN_SIZE) + MIN_BIN_SIZE
    slopes_offset = jnp.log(jnp.exp(1.0 - MIN_SLOPE) - 1.0)
    slopes = jax.nn.softplus(params[2 * K:] + slopes_offset) + MIN_SLOPE
    x_pos = jnp.concatenate([jnp.array([0.0]), jnp.cumsum(widths)]) + RANGE_MIN
    y_pos = jnp.concatenate([jnp.array([0.0]), jnp.cumsum(heights)]) + RANGE_MIN

    def padto(a):
        return jnp.pad(a, (0, TPAD - a.shape[0]), constant_values=1.0).astype(jnp.float32)

    invw = 1.0 / (x_pos[1:] - x_pos[:-1])
    h = y_pos[1:] - y_pos[:-1]
    s_tab = h * invw
    xpos_p = padto(x_pos)
    ypos_p = padto(y_pos)
    d_p = padto(slopes)
    invw_p = padto(invw)
    h_p = padto(h)
    t0_p = padto(-x_pos[:128] * invw)
    xposc_p = x_pos[0:128:8].astype(jnp.float32)
    c0_p = padto(slopes[1:] + slopes[:-1] - 2.0 * s_tab)
    ln_p = jnp.asarray(np.log1p(np.arange(128) / 128.0) - 127.0 * np.log(2.0),
                       dtype=jnp.float32)
    rcp_p = jnp.asarray(2.0 ** -23 / (1.0 + np.arange(128) / 128.0),
                        dtype=jnp.float32)

    mesh = plsc.VectorSubcoreMesh(core_axis_name="c", subcore_axis_name="s")
    f32 = jnp.float32
    run = pl.kernel(
        _sc_body,
        mesh=mesh,
        compiler_params=pltpu.CompilerParams(needs_layout_passes=False),
        out_type=(jax.ShapeDtypeStruct((N,), f32),
                  jax.ShapeDtypeStruct((N,), f32)),
        scratch_types=[
            pltpu.VMEM((TPAD,), f32),
            pltpu.VMEM((TPAD,), f32),
            pltpu.VMEM((TPAD,), f32),
            pltpu.VMEM((TPAD,), f32),
            pltpu.VMEM((TPAD,), f32),
            pltpu.VMEM((TPAD,), f32),
            pltpu.VMEM((16,), f32),
            pltpu.VMEM((TPAD,), f32),
            pltpu.VMEM((128,), f32),
            pltpu.VMEM((128,), f32),
            pltpu.VMEM((CHUNK,), f32),
            pltpu.VMEM((CHUNK,), f32),
            pltpu.VMEM((CHUNK,), f32),
            pltpu.VMEM((CHUNK,), f32),
            pltpu.VMEM((CHUNK,), f32),
            pltpu.VMEM((CHUNK,), f32),
            pltpu.SemaphoreType.DMA,
            pltpu.SemaphoreType.DMA,
            pltpu.SemaphoreType.DMA,
            pltpu.SemaphoreType.DMA,
            pltpu.SemaphoreType.DMA,
            pltpu.SemaphoreType.DMA,
        ],
    )
    return run(x, xpos_p, ypos_p, d_p, invw_p, h_p, t0_p,
               xposc_p, c0_p, ln_p, rcp_p)


# speedup vs baseline: 1.0734x; 1.0734x over previous
import functools

import jax
import jax.numpy as jnp
import numpy as np
from jax import lax
from jax.experimental import pallas as pl
from jax.experimental.pallas import tpu as pltpu
from jax.experimental.pallas import tpu_sc as plsc

RANGE_MIN = -5.0
RANGE_MAX = 5.0
MIN_BIN_SIZE = 0.0001
MIN_SLOPE = 0.0001

LN2 = 0.6931471805599453
SQRT2 = 1.4142135623730951

N = 8388608
NC, NS, L = 2, 16, 16
NW = NC * NS
PER_W = N // NW
CHUNK = 16384
N_CHUNKS = PER_W // CHUNK
N_STEPS = N_CHUNKS // 2
VREGS = CHUNK // L
TPAD = 144


def _log_approx(t):
    bits = lax.bitcast_convert_type(t, jnp.int32)
    e_i = (bits >> 23) - 127
    m = lax.bitcast_convert_type((bits & 0x007FFFFF) | 0x3F800000, jnp.float32)
    big = m >= SQRT2
    m = jnp.where(big, m * 0.5, m)
    e_f = e_i.astype(jnp.float32) + jnp.where(big, 1.0, 0.0)
    z = (m - 1.0) / (m + 1.0)
    z2 = z * z
    p = z * (2.0 + z2 * (2.0 / 3.0 + z2 * (2.0 / 5.0 + z2 * (2.0 / 7.0))))
    return e_f * LN2 + p


def _sc_body(x_hbm, xpos_hbm, ypos_hbm, d_hbm, invw_hbm, h_hbm, t0_hbm,
             xposc_hbm, c0_hbm, ln_hbm, rcp_hbm,
             y_hbm, ld_hbm,
             xpos_v, ypos_v, d_v, invw_v, h_v, t0_v, xposc_v, c0_v, ln_v, rcp_v,
             x0, x1, y0, y1, l0, l1,
             sem_in0, sem_in1, sem_oy0, sem_oy1, sem_ol0, sem_ol1):
    wid = lax.axis_index("s") * NC + lax.axis_index("c")
    base = wid * PER_W

    pltpu.sync_copy(xpos_hbm, xpos_v)
    pltpu.sync_copy(ypos_hbm, ypos_v)
    pltpu.sync_copy(d_hbm, d_v)
    pltpu.sync_copy(invw_hbm, invw_v)
    pltpu.sync_copy(h_hbm, h_v)
    pltpu.sync_copy(t0_hbm, t0_v)
    pltpu.sync_copy(xposc_hbm, xposc_v)
    pltpu.sync_copy(c0_hbm, c0_v)
    pltpu.sync_copy(ln_hbm, ln_v)
    pltpu.sync_copy(rcp_hbm, rcp_v)

    coarse = xposc_v[pl.ds(0, L)]
    fine4 = xposc_v[pl.ds(L, L)]
    fine2a = xposc_v[pl.ds(2 * L, L)]
    fine2b = xposc_v[pl.ds(3 * L, L)]

    zero_f = jnp.zeros((L,), jnp.float32)
    one_f = jnp.full((L,), 1.0, jnp.float32)
    rmin_f = jnp.full((L,), RANGE_MIN, jnp.float32)
    rmax_f = jnp.full((L,), RANGE_MAX, jnp.float32)
    ln2_f = jnp.full((L,), LN2, jnp.float32)
    zero_i = jnp.zeros((L,), jnp.int32)
    m7f_i = jnp.full((L,), 0x7F, jnp.int32)
    mffff_i = jnp.full((L,), 0xFFFF, jnp.int32)

    def make_vreg_body(x_v, y_v, ld_v):
      def vreg_body(off):
        xv = x_v[pl.ds(off, L)]
        c = zero_i
        for step in (8, 4, 2, 1):
            cand = c + step
            knot = jnp.take_along_axis(coarse, cand, axis=0)
            c = jnp.where(knot <= xv, cand, c)
        k5 = jnp.take_along_axis(fine4, c, axis=0)
        m5 = k5 <= xv
        k6 = jnp.where(m5, jnp.take_along_axis(fine2b, c, axis=0),
                       jnp.take_along_axis(fine2a, c, axis=0))
        m6 = k6 <= xv
        b = c * 8 + jnp.where(m5, 4, 0) + jnp.where(m6, 2, 0)
        cand = b + 1
        knot = plsc.load_gather(xpos_v, [cand])
        b = jnp.where(knot <= xv, cand, b)
        t0 = plsc.load_gather(t0_v, [b])
        y_k = plsc.load_gather(ypos_v, [b])
        iw = plsc.load_gather(invw_v, [b])
        hh = plsc.load_gather(h_v, [b])
        d_k = plsc.load_gather(d_v, [b])
        d_k1 = plsc.load_gather(d_v, [b + 1])
        c0 = plsc.load_gather(c0_v, [b])
        s_ = hh * iw
        xi = jnp.minimum(jnp.maximum(xv * iw + t0, zero_f), one_f)
        om = one_f - xi
        xiom = xi * om
        dkom = d_k * om
        num = xi * (s_ * xi + dkom)
        den = s_ + c0 * xiom
        rden = 1.0 / den
        y_spline = y_k + hh * (num * rden)
        numd = s_ * s_ * (d_k1 * xi * xi + (s_ + s_) * xiom + dkom * om)
        deriv = numd * (rden * rden)
        below = xv < rmin_f
        above = xv > rmax_f
        yv = jnp.where(below, (xv - rmin_f) * d_k + rmin_f,
                       jnp.where(above, (xv - rmax_f) * d_k1 + rmax_f,
                                 y_spline))
        bits = lax.bitcast_convert_type(deriv, jnp.int32)
        ubits = lax.bitcast_convert_type(deriv, jnp.uint32)
        e_f = (bits >> 23).astype(jnp.float32)
        j = lax.convert_element_type((ubits << 9) >> 25, jnp.int32)
        f_cvt = ((ubits << 16) >> 16).astype(jnp.float32)
        lnm = plsc.load_gather(ln_v, [j]) + f_cvt * plsc.load_gather(rcp_v, [j])
        y_v[pl.ds(off, L)] = yv
        ld_v[pl.ds(off, L)] = e_f * ln2_f + lnm
      return vreg_body

    def half(i, g, x_v, y_v, ld_v, sem_in, sem_oy, sem_ol):
        lo = base + g * CHUNK
        out_y = pltpu.make_async_copy(y_v, y_hbm.at[pl.ds(lo, CHUNK)], sem_oy)
        out_l = pltpu.make_async_copy(ld_v, ld_hbm.at[pl.ds(lo, CHUNK)], sem_ol)

        @pl.when(i > 0)
        def _():
            out_y.wait()
            out_l.wait()

        pltpu.make_async_copy(x_hbm.at[pl.ds(lo, CHUNK)], x_v, sem_in).wait()
        plsc.parallel_loop(0, CHUNK, L, unroll=16)(make_vreg_body(x_v, y_v, ld_v))
        out_y.start()
        out_l.start()

        @pl.when(i < N_STEPS - 1)
        def _():
            nxt = lo + 2 * CHUNK
            pltpu.make_async_copy(x_hbm.at[pl.ds(nxt, CHUNK)], x_v, sem_in).start()

    pltpu.make_async_copy(x_hbm.at[pl.ds(base, CHUNK)], x0, sem_in0).start()
    pltpu.make_async_copy(x_hbm.at[pl.ds(base + CHUNK, CHUNK)], x1, sem_in1).start()

    def step(i, carry):
        half(i, 2 * i, x0, y0, l0, sem_in0, sem_oy0, sem_ol0)
        half(i, 2 * i + 1, x1, y1, l1, sem_in1, sem_oy1, sem_ol1)
        return carry

    lax.fori_loop(0, N_STEPS, step, 0)

    tail = base + (N_CHUNKS - 2) * CHUNK
    pltpu.make_async_copy(y0, y_hbm.at[pl.ds(tail, CHUNK)], sem_oy0).wait()
    pltpu.make_async_copy(l0, ld_hbm.at[pl.ds(tail, CHUNK)], sem_ol0).wait()
    pltpu.make_async_copy(y1, y_hbm.at[pl.ds(tail + CHUNK, CHUNK)], sem_oy1).wait()
    pltpu.make_async_copy(l1, ld_hbm.at[pl.ds(tail + CHUNK, CHUNK)], sem_ol1).wait()


@jax.jit
def kernel(x, params):
    K = (params.shape[-1] - 1) // 3
    total_size = RANGE_MAX - RANGE_MIN
    widths = jax.nn.softmax(params[:K]) * (total_size - K * MIN_BIN_SIZE) + MIN_BIN_SIZE
    heights = jax.nn.softmax(params[K:2 * K]) * (total_size - K * MIN_BIN_SIZE) + MIN_BIN_SIZE
    slopes_offset = jnp.log(jnp.exp(1.0 - MIN_SLOPE) - 1.0)
    slopes = jax.nn.softplus(params[2 * K:] + slopes_offset) + MIN_SLOPE
    x_pos = jnp.concatenate([jnp.array([0.0]), jnp.cumsum(widths)]) + RANGE_MIN
    y_pos = jnp.concatenate([jnp.array([0.0]), jnp.cumsum(heights)]) + RANGE_MIN

    def padto(a):
        return jnp.pad(a, (0, TPAD - a.shape[0]), constant_values=1.0).astype(jnp.float32)

    invw = 1.0 / (x_pos[1:] - x_pos[:-1])
    h = y_pos[1:] - y_pos[:-1]
    s_tab = h * invw
    xpos_p = padto(x_pos)
    ypos_p = padto(y_pos)
    d_p = padto(slopes)
    invw_p = padto(invw)
    h_p = padto(h)
    t0_p = padto(-x_pos[:128] * invw)
    xposc_p = jnp.concatenate([x_pos[0:128:8], x_pos[4:128:8],
                               x_pos[2:128:8], x_pos[6:128:8]]).astype(jnp.float32)
    c0_p = padto(slopes[1:] + slopes[:-1] - 2.0 * s_tab)
    ln_p = jnp.asarray(np.log1p(np.arange(128) / 128.0) - 127.0 * np.log(2.0),
                       dtype=jnp.float32)
    rcp_p = jnp.asarray(2.0 ** -23 / (1.0 + np.arange(128) / 128.0),
                        dtype=jnp.float32)

    mesh = plsc.VectorSubcoreMesh(core_axis_name="c", subcore_axis_name="s")
    f32 = jnp.float32
    run = pl.kernel(
        _sc_body,
        mesh=mesh,
        compiler_params=pltpu.CompilerParams(needs_layout_passes=False),
        out_type=(jax.ShapeDtypeStruct((N,), f32),
                  jax.ShapeDtypeStruct((N,), f32)),
        scratch_types=[
            pltpu.VMEM((TPAD,), f32),
            pltpu.VMEM((TPAD,), f32),
            pltpu.VMEM((TPAD,), f32),
            pltpu.VMEM((TPAD,), f32),
            pltpu.VMEM((TPAD,), f32),
            pltpu.VMEM((TPAD,), f32),
            pltpu.VMEM((64,), f32),
            pltpu.VMEM((TPAD,), f32),
            pltpu.VMEM((128,), f32),
            pltpu.VMEM((128,), f32),
            pltpu.VMEM((CHUNK,), f32),
            pltpu.VMEM((CHUNK,), f32),
            pltpu.VMEM((CHUNK,), f32),
            pltpu.VMEM((CHUNK,), f32),
            pltpu.VMEM((CHUNK,), f32),
            pltpu.VMEM((CHUNK,), f32),
            pltpu.SemaphoreType.DMA,
            pltpu.SemaphoreType.DMA,
            pltpu.SemaphoreType.DMA,
            pltpu.SemaphoreType.DMA,
            pltpu.SemaphoreType.DMA,
            pltpu.SemaphoreType.DMA,
        ],
    )
    return run(x, xpos_p, ypos_p, d_p, invw_p, h_p, t0_p,
               xposc_p, c0_p, ln_p, rcp_p)


# speedup vs baseline: 1.1000x; 1.0248x over previous
import functools

import jax
import jax.numpy as jnp
import numpy as np
from jax import lax
from jax.experimental import pallas as pl
from jax.experimental.pallas import tpu as pltpu
from jax.experimental.pallas import tpu_sc as plsc

RANGE_MIN = -5.0
RANGE_MAX = 5.0
MIN_BIN_SIZE = 0.0001
MIN_SLOPE = 0.0001

LN2 = 0.6931471805599453
SQRT2 = 1.4142135623730951

N = 8388608
NC, NS, L = 2, 16, 16
NW = NC * NS
PER_W = N // NW
CHUNK = 16384
N_CHUNKS = PER_W // CHUNK
N_STEPS = N_CHUNKS // 2
VREGS = CHUNK // L
TPAD = 144


def _log_approx(t):
    bits = lax.bitcast_convert_type(t, jnp.int32)
    e_i = (bits >> 23) - 127
    m = lax.bitcast_convert_type((bits & 0x007FFFFF) | 0x3F800000, jnp.float32)
    big = m >= SQRT2
    m = jnp.where(big, m * 0.5, m)
    e_f = e_i.astype(jnp.float32) + jnp.where(big, 1.0, 0.0)
    z = (m - 1.0) / (m + 1.0)
    z2 = z * z
    p = z * (2.0 + z2 * (2.0 / 3.0 + z2 * (2.0 / 5.0 + z2 * (2.0 / 7.0))))
    return e_f * LN2 + p


def _sc_body(x_hbm, xpos_hbm, ypos_hbm, d_hbm, invw_hbm, h_hbm, t0_hbm,
             xposc_hbm, c0_hbm, ln_hbm, rcp_hbm,
             y_hbm, ld_hbm,
             xpos_v, ypos_v, d_v, invw_v, h_v, t0_v, xposc_v, c0_v, ln_v, rcp_v,
             x0, x1, y0, y1, l0, l1,
             sem_in0, sem_in1, sem_oy0, sem_oy1, sem_ol0, sem_ol1):
    wid = lax.axis_index("s") * NC + lax.axis_index("c")
    base = wid * PER_W

    pltpu.sync_copy(xpos_hbm, xpos_v)
    pltpu.sync_copy(ypos_hbm, ypos_v)
    pltpu.sync_copy(d_hbm, d_v)
    pltpu.sync_copy(invw_hbm, invw_v)
    pltpu.sync_copy(h_hbm, h_v)
    pltpu.sync_copy(t0_hbm, t0_v)
    pltpu.sync_copy(xposc_hbm, xposc_v)
    pltpu.sync_copy(c0_hbm, c0_v)
    pltpu.sync_copy(ln_hbm, ln_v)
    pltpu.sync_copy(rcp_hbm, rcp_v)

    coarse = xposc_v[pl.ds(0, L)]
    fine4 = xposc_v[pl.ds(L, L)]
    fine2a = xposc_v[pl.ds(2 * L, L)]
    fine2b = xposc_v[pl.ds(3 * L, L)]

    zero_f = jnp.zeros((L,), jnp.float32)
    one_f = jnp.full((L,), 1.0, jnp.float32)
    rmin_f = jnp.full((L,), RANGE_MIN, jnp.float32)
    rmax_f = jnp.full((L,), RANGE_MAX, jnp.float32)
    ln2_f = jnp.full((L,), LN2, jnp.float32)
    zero_i = jnp.zeros((L,), jnp.int32)
    m7f_i = jnp.full((L,), 0x7F, jnp.int32)
    mffff_i = jnp.full((L,), 0xFFFF, jnp.int32)

    def make_vreg_body(x_v, y_v, ld_v):
      def vreg_body(off):
        xv = x_v[pl.ds(off, L)]
        xc = jnp.minimum(jnp.maximum(xv, rmin_f), rmax_f)
        c = zero_i
        for step in (8, 4, 2, 1):
            cand = c + step
            knot = jnp.take_along_axis(coarse, cand, axis=0)
            c = jnp.where(knot <= xc, cand, c)
        k5 = jnp.take_along_axis(fine4, c, axis=0)
        m5 = k5 <= xc
        k6 = jnp.where(m5, jnp.take_along_axis(fine2b, c, axis=0),
                       jnp.take_along_axis(fine2a, c, axis=0))
        m6 = k6 <= xc
        b = c * 8 + jnp.where(m5, 4, 0) + jnp.where(m6, 2, 0)
        cand = b + 1
        knot = plsc.load_gather(xpos_v, [cand])
        b = jnp.where(knot <= xc, cand, b)
        t0 = plsc.load_gather(t0_v, [b])
        y_k = plsc.load_gather(ypos_v, [b])
        iw = plsc.load_gather(invw_v, [b])
        hh = plsc.load_gather(h_v, [b])
        d_k = plsc.load_gather(d_v, [b])
        d_k1 = plsc.load_gather(d_v, [b + 1])
        c0 = plsc.load_gather(c0_v, [b])
        s_ = hh * iw
        xi = jnp.minimum(jnp.maximum(xc * iw + t0, zero_f), one_f)
        om = one_f - xi
        xiom = xi * om
        dkom = d_k * om
        num = xi * (s_ * xi + dkom)
        den = s_ + c0 * xiom
        rden = 1.0 / den
        y_spline = y_k + hh * (num * rden)
        numd = s_ * s_ * (d_k1 * xi * xi + (s_ + s_) * xiom + dkom * om)
        deriv = numd * (rden * rden)
        d_e = jnp.where(xv >= zero_f, d_k1, d_k)
        yv = y_spline + (xv - xc) * d_e
        bits = lax.bitcast_convert_type(deriv, jnp.int32)
        ubits = lax.bitcast_convert_type(deriv, jnp.uint32)
        e_f = (bits >> 23).astype(jnp.float32)
        j = lax.convert_element_type((ubits << 9) >> 25, jnp.int32)
        f_cvt = ((ubits << 16) >> 16).astype(jnp.float32)
        lnm = plsc.load_gather(ln_v, [j]) + f_cvt * plsc.load_gather(rcp_v, [j])
        y_v[pl.ds(off, L)] = yv
        ld_v[pl.ds(off, L)] = e_f * ln2_f + lnm
      return vreg_body

    def half(i, g, x_v, y_v, ld_v, sem_in, sem_oy, sem_ol):
        lo = base + g * CHUNK
        out_y = pltpu.make_async_copy(y_v, y_hbm.at[pl.ds(lo, CHUNK)], sem_oy)
        out_l = pltpu.make_async_copy(ld_v, ld_hbm.at[pl.ds(lo, CHUNK)], sem_ol)

        @pl.when(i > 0)
        def _():
            out_y.wait()
            out_l.wait()

        pltpu.make_async_copy(x_hbm.at[pl.ds(lo, CHUNK)], x_v, sem_in).wait()
        plsc.parallel_loop(0, CHUNK, L, unroll=16)(make_vreg_body(x_v, y_v, ld_v))
        out_y.start()
        out_l.start()

        @pl.when(i < N_STEPS - 1)
        def _():
            nxt = lo + 2 * CHUNK
            pltpu.make_async_copy(x_hbm.at[pl.ds(nxt, CHUNK)], x_v, sem_in).start()

    pltpu.make_async_copy(x_hbm.at[pl.ds(base, CHUNK)], x0, sem_in0).start()
    pltpu.make_async_copy(x_hbm.at[pl.ds(base + CHUNK, CHUNK)], x1, sem_in1).start()

    def step(i, carry):
        half(i, 2 * i, x0, y0, l0, sem_in0, sem_oy0, sem_ol0)
        half(i, 2 * i + 1, x1, y1, l1, sem_in1, sem_oy1, sem_ol1)
        return carry

    lax.fori_loop(0, N_STEPS, step, 0)

    tail = base + (N_CHUNKS - 2) * CHUNK
    pltpu.make_async_copy(y0, y_hbm.at[pl.ds(tail, CHUNK)], sem_oy0).wait()
    pltpu.make_async_copy(l0, ld_hbm.at[pl.ds(tail, CHUNK)], sem_ol0).wait()
    pltpu.make_async_copy(y1, y_hbm.at[pl.ds(tail + CHUNK, CHUNK)], sem_oy1).wait()
    pltpu.make_async_copy(l1, ld_hbm.at[pl.ds(tail + CHUNK, CHUNK)], sem_ol1).wait()


@jax.jit
def kernel(x, params):
    K = (params.shape[-1] - 1) // 3
    total_size = RANGE_MAX - RANGE_MIN
    widths = jax.nn.softmax(params[:K]) * (total_size - K * MIN_BIN_SIZE) + MIN_BIN_SIZE
    heights = jax.nn.softmax(params[K:2 * K]) * (total_size - K * MIN_BIN_SIZE) + MIN_BIN_SIZE
    slopes_offset = jnp.log(jnp.exp(1.0 - MIN_SLOPE) - 1.0)
    slopes = jax.nn.softplus(params[2 * K:] + slopes_offset) + MIN_SLOPE
    x_pos = jnp.concatenate([jnp.array([0.0]), jnp.cumsum(widths)]) + RANGE_MIN
    y_pos = jnp.concatenate([jnp.array([0.0]), jnp.cumsum(heights)]) + RANGE_MIN

    def padto(a):
        return jnp.pad(a, (0, TPAD - a.shape[0]), constant_values=1.0).astype(jnp.float32)

    invw = 1.0 / (x_pos[1:] - x_pos[:-1])
    h = y_pos[1:] - y_pos[:-1]
    s_tab = h * invw
    xpos_p = padto(x_pos)
    ypos_p = padto(y_pos)
    d_p = padto(slopes)
    invw_p = padto(invw)
    h_p = padto(h)
    t0_p = padto(-x_pos[:128] * invw)
    xposc_p = jnp.concatenate([x_pos[0:128:8], x_pos[4:128:8],
                               x_pos[2:128:8], x_pos[6:128:8]]).astype(jnp.float32)
    c0_p = padto(slopes[1:] + slopes[:-1] - 2.0 * s_tab)
    ln_p = jnp.asarray(np.log1p(np.arange(128) / 128.0) - 127.0 * np.log(2.0),
                       dtype=jnp.float32)
    rcp_p = jnp.asarray(2.0 ** -23 / (1.0 + np.arange(128) / 128.0),
                        dtype=jnp.float32)

    mesh = plsc.VectorSubcoreMesh(core_axis_name="c", subcore_axis_name="s")
    f32 = jnp.float32
    run = pl.kernel(
        _sc_body,
        mesh=mesh,
        compiler_params=pltpu.CompilerParams(needs_layout_passes=False),
        out_type=(jax.ShapeDtypeStruct((N,), f32),
                  jax.ShapeDtypeStruct((N,), f32)),
        scratch_types=[
            pltpu.VMEM((TPAD,), f32),
            pltpu.VMEM((TPAD,), f32),
            pltpu.VMEM((TPAD,), f32),
            pltpu.VMEM((TPAD,), f32),
            pltpu.VMEM((TPAD,), f32),
            pltpu.VMEM((TPAD,), f32),
            pltpu.VMEM((64,), f32),
            pltpu.VMEM((TPAD,), f32),
            pltpu.VMEM((128,), f32),
            pltpu.VMEM((128,), f32),
            pltpu.VMEM((CHUNK,), f32),
            pltpu.VMEM((CHUNK,), f32),
            pltpu.VMEM((CHUNK,), f32),
            pltpu.VMEM((CHUNK,), f32),
            pltpu.VMEM((CHUNK,), f32),
            pltpu.VMEM((CHUNK,), f32),
            pltpu.SemaphoreType.DMA,
            pltpu.SemaphoreType.DMA,
            pltpu.SemaphoreType.DMA,
            pltpu.SemaphoreType.DMA,
            pltpu.SemaphoreType.DMA,
            pltpu.SemaphoreType.DMA,
        ],
    )
    return run(x, xpos_p, ypos_p, d_p, invw_p, h_p, t0_p,
               xposc_p, c0_p, ln_p, rcp_p)
